# baseline (speedup 1.0000x reference)
import jax
import jax.numpy as jnp
from jax import lax
from jax.experimental import pallas as pl
from jax.experimental.pallas import tpu as pltpu

B, N, NS = 8, 512, 16
C = 256
S = 8
CH = C // S
NCHUNK = 4
PC = N // NCHUNK
RC = PC * NS
CNT = float(B * N * NS)
EPS = 1e-5


def _onehot_rows(idxc):
    tgt = lax.broadcasted_iota(jnp.int32, (PC, NS, N), 2).astype(jnp.float32)
    sel = jnp.where(idxc[:, :, None] == tgt, 1.0, 0.0)
    return sel.reshape(RC, N)


def _rep_rows(a):
    d = a.shape[-1]
    return jnp.broadcast_to(a[:, None, :], (PC, NS, d)).reshape(RC, d)


def _scale_shift(s_s, ss_s, gamma, beta):
    mean = s_s[...] / CNT
    var = ss_s[...] / CNT - mean * mean
    scale = gamma / jnp.sqrt(var + EPS)
    shift = beta - mean * scale
    return scale, shift


def _accum(first, s_ref, ss_ref, t, axis):
    s_new = jnp.sum(t, axis=axis, keepdims=True)
    ss_new = jnp.sum(t * t, axis=axis, keepdims=True)
    zero = jnp.zeros_like(s_new)
    s_ref[...] = jnp.where(first, zero, s_ref[...]) + s_new
    ss_ref[...] = jnp.where(first, zero, ss_ref[...]) + ss_new


def _dg(a, b, ca, cb):
    return lax.dot_general(a, b, (((ca,), (cb,)), ((), ())),
                           preferred_element_type=jnp.float32)


def _body(xt_ref, p_ref, pT_ref,
          wqT, bq, wkT, bk, wvT, bv, wp0T, bp0, gp_r, betap_r, wp1T, bp1,
          gw0_r, bw0_r, ww0T, bw0l_c, gw1_c, bw1_c, ww1T, bw1l,
          out_ref,
          q_s, k_s, v_s, idx_s, hT_s,
          sp_s, ssp_s, s0_s, ss0_s, s1_s, ss1_s):
    s = pl.program_id(0)
    b = pl.program_id(1)
    c = pl.program_id(2)
    first = jnp.logical_and(b == 0, c == 0)
    nb = pl.ds(pl.multiple_of(c * PC, PC), PC)

    def pr0_feat(sel, pc, pT):
        pw = _dg(pT, wp0T[...], 0, 0)
        pwc = jnp.dot(pc, wp0T[...], preferred_element_type=jnp.float32)
        return _dg(sel, pw, 1, 0) - _rep_rows(pwc) + bp0[...]

    def pos_feat(sel, pc, pT):
        scalep, shiftp = _scale_shift(sp_s, ssp_s, gp_r[...], betap_r[...])
        pr = jnp.maximum(pr0_feat(sel, pc, pT) * scalep + shiftp, 0.0)
        return jnp.dot(pr, wp1T[...], preferred_element_type=jnp.float32) + bp1[...]

    def w0_feat(sel, p_r):
        g_k = jnp.dot(sel, k_s[b], preferred_element_type=jnp.float32)
        return g_k - _rep_rows(q_s[b, nb, :]) + p_r

    @pl.when(s == 0)
    def _pass0():
        xtc = xt_ref[0]
        q_s[b, nb, :] = jnp.dot(xtc, wqT[...], preferred_element_type=jnp.float32) + bq[...]
        k_s[b, nb, :] = jnp.dot(xtc, wkT[...], preferred_element_type=jnp.float32) + bk[...]
        v_s[b, nb, :] = jnp.dot(xtc, wvT[...], preferred_element_type=jnp.float32) + bv[...]

        pc = p_ref[0]
        pT = pT_ref[0]
        dx = pc[:, 0:1] - pT[0:1, :]
        dy = pc[:, 1:2] - pT[1:2, :]
        dz = pc[:, 2:3] - pT[2:3, :]
        d2 = (dx * dx + dy * dy) + dz * dz

        colid = lax.broadcasted_iota(jnp.int32, (PC, N), 1).astype(jnp.float32)
        work = d2
        cols = []
        for _ in range(NS):
            m = jnp.min(work, axis=1, keepdims=True)
            cand = jnp.where(work == m, colid, jnp.float32(N))
            am = jnp.min(cand, axis=1, keepdims=True)
            cols.append(am)
            work = jnp.where(colid == am, jnp.inf, work)
        idxc = jnp.concatenate(cols, axis=1)
        idx_s[b, c] = idxc

        sel = _onehot_rows(idxc)
        pr0 = pr0_feat(sel, pc, pT)
        _accum(first, sp_s, ssp_s, pr0, 0)

    @pl.when(s == 1)
    def _pass1():
        sel = _onehot_rows(idx_s[b, c])
        w0 = w0_feat(sel, pos_feat(sel, p_ref[0], pT_ref[0]))
        _accum(first, s0_s, ss0_s, w0, 0)

    @pl.when(s == 2)
    def _pass2():
        sel = _onehot_rows(idx_s[b, c])
        w0 = w0_feat(sel, pos_feat(sel, p_ref[0], pT_ref[0]))
        scale0, shift0 = _scale_shift(s0_s, ss0_s, gw0_r[...], bw0_r[...])
        w0n = jnp.maximum(w0 * scale0 + shift0, 0.0)
        hT = _dg(ww0T[...], w0n, 0, 1) + bw0l_c[...]
        hT_s[b, c] = hT
        _accum(first, s1_s, ss1_s, hT, 1)

    @pl.when(s == 3)
    def _pass3():
        scale1, shift1 = _scale_shift(s1_s, ss1_s, gw1_c[...], bw1_c[...])
        hnT = jnp.maximum(hT_s[b, c] * scale1 + shift1, 0.0)
        w1 = _dg(hnT, ww1T[...], 0, 0) + bw1l[...]
        w3 = w1.reshape(PC, NS, CH)
        mx = jnp.max(w3, axis=1, keepdims=True)
        e = jnp.exp(w3 - mx)
        rinv = 1.0 / jnp.sum(e, axis=1, keepdims=True)
        wt = jnp.concatenate([e] * S, axis=2)

        sel = _onehot_rows(idx_s[b, c])
        p_r = pos_feat(sel, p_ref[0], pT_ref[0])
        g_v = jnp.dot(sel, v_s[b], preferred_element_type=jnp.float32)
        a = (g_v + p_r).reshape(PC, NS, C)
        acc = jnp.sum(a * wt, axis=1)
        rt = jnp.concatenate([rinv[:, 0, :]] * S, axis=1)
        out_ref[0] = acc * rt


def kernel(p, x, Wq, bq, Wk, bk, Wv, bv, Wp0, bp0, gp, betap, Wp1, bp1,
           gw0, bw0, Ww0, bw0l, gw1, bw1, Ww1, bw1l):
    f32 = jnp.float32
    xt = jnp.transpose(x, (0, 2, 1))
    pT = jnp.transpose(p, (0, 2, 1))
    wqT, wkT, wvT = Wq.T, Wk.T, Wv.T
    wp0T, wp1T, ww0T, ww1T = Wp0.T, Wp1.T, Ww0.T, Ww1.T
    r2 = lambda a: a.reshape(1, -1)
    rc = lambda a: a.reshape(-1, 1)
    bq2, bk2, bv2, bp02, bp12, bw1l2 = map(r2, (bq, bk, bv, bp0, bp1, bw1l))
    gp2, betap2, gw02, bw02 = map(r2, (gp, betap, gw0, bw0))
    bw0lc, gw1c, bw1c = map(rc, (bw0l, gw1, bw1))

    grid = (4, B, NCHUNK)
    full = lambda shape: pl.BlockSpec(shape, lambda s, b, c: (0,) * len(shape))
    bc = lambda *shape: pl.BlockSpec((1,) + shape,
                                     lambda s, b, c: (b, c) + (0,) * (len(shape) - 1))
    bonly = lambda *shape: pl.BlockSpec((1,) + shape,
                                        lambda s, b, c: (b,) + (0,) * len(shape))

    out = pl.pallas_call(
        _body,
        grid=grid,
        in_specs=[bc(PC, C), bc(PC, 3), bonly(3, N),
                  full((C, C)), full((1, C)), full((C, C)), full((1, C)),
                  full((C, C)), full((1, C)), full((3, 3)), full((1, 3)),
                  full((1, 3)), full((1, 3)), full((3, C)), full((1, C)),
                  full((1, C)), full((1, C)), full((C, CH)), full((CH, 1)),
                  full((CH, 1)), full((CH, 1)), full((CH, CH)), full((1, CH))],
        out_specs=pl.BlockSpec(
            (1, PC, C),
            lambda s, b, c: (jnp.where(s == 3, b, 0), jnp.where(s == 3, c, 0), 0)),
        out_shape=jax.ShapeDtypeStruct((B, N, C), f32),
        scratch_shapes=[
            pltpu.VMEM((B, N, C), f32),
            pltpu.VMEM((B, N, C), f32),
            pltpu.VMEM((B, N, C), f32),
            pltpu.VMEM((B, NCHUNK, PC, NS), f32),
            pltpu.VMEM((B, NCHUNK, CH, RC), f32),
            pltpu.VMEM((1, 3), f32), pltpu.VMEM((1, 3), f32),
            pltpu.VMEM((1, C), f32), pltpu.VMEM((1, C), f32),
            pltpu.VMEM((CH, 1), f32), pltpu.VMEM((CH, 1), f32),
        ],
    )(xt, p, pT, wqT, bq2, wkT, bk2, wvT, bv2, wp0T, bp02, gp2, betap2,
      wp1T, bp12, gw02, bw02, ww0T, bw0lc, gw1c, bw1c, ww1T, bw1l2)

    return out


# speedup vs baseline: 1.2089x; 1.2089x over previous
import jax
import jax.numpy as jnp
from jax import lax
from jax.experimental import pallas as pl

B, N, NS = 8, 512, 16
C = 256
S = 8
CH = C // S
RC = N * NS
CNT = float(B * N * NS)
EPS = 1e-5


def _onehot_rows(idxc):
    tgt = lax.broadcasted_iota(jnp.int32, (N, NS, N), 2).astype(jnp.float32)
    sel = jnp.where(idxc[:, :, None] == tgt, 1.0, 0.0)
    return sel.reshape(RC, N)


def _rep_rows(a):
    d = a.shape[-1]
    return jnp.broadcast_to(a[:, None, :], (N, NS, d)).reshape(RC, d)


def _bn_scale_shift(s_ref, ss_ref, gamma, beta):
    ssum = jnp.sum(s_ref[...], axis=0)
    sssum = jnp.sum(ss_ref[...], axis=0)
    mean = ssum / CNT
    var = sssum / CNT - mean * mean
    scale = gamma / jnp.sqrt(var + EPS)
    shift = beta - mean * scale
    return scale, shift


def _dg(a, b, ca, cb):
    return lax.dot_general(a, b, (((ca,), (cb,)), ((), ())),
                           preferred_element_type=jnp.float32)


def _pr0_feat(sel, pc, pT, wp0T_r, bp0_r):
    pw = _dg(pT, wp0T_r[...], 0, 0)
    pwc = jnp.dot(pc, wp0T_r[...], preferred_element_type=jnp.float32)
    return _dg(sel, pw, 1, 0) - _rep_rows(pwc) + bp0_r[...]


def _pos_feat(sel, p_ref, pT_ref, sp_ref, ssp_ref, gp_r, betap_r,
              wp0T_r, bp0_r, wp1T_r, bp1_r):
    scalep, shiftp = _bn_scale_shift(sp_ref, ssp_ref, gp_r[...], betap_r[...])
    pr0 = _pr0_feat(sel, p_ref[0], pT_ref[0], wp0T_r, bp0_r)
    pr = jnp.maximum(pr0 * scalep + shiftp, 0.0)
    return jnp.dot(pr, wp1T_r[...], preferred_element_type=jnp.float32) + bp1_r[...]


def _k1_body(xt_ref, p_ref, pT_ref, wqT, bq, wkT, bk, wvT, bv, wp0T, bp0,
             q_ref, k_ref, v_ref, idx_ref, sp_ref, ssp_ref):
    xtc = xt_ref[0]
    q_ref[0] = jnp.dot(xtc, wqT[...], preferred_element_type=jnp.float32) + bq[...]
    k_ref[0] = jnp.dot(xtc, wkT[...], preferred_element_type=jnp.float32) + bk[...]
    v_ref[0] = jnp.dot(xtc, wvT[...], preferred_element_type=jnp.float32) + bv[...]

    pc = p_ref[0]
    pT = pT_ref[0]
    dx = pc[:, 0:1] - pT[0:1, :]
    dy = pc[:, 1:2] - pT[1:2, :]
    dz = pc[:, 2:3] - pT[2:3, :]
    d2 = (dx * dx + dy * dy) + dz * dz

    colid = lax.broadcasted_iota(jnp.int32, (N, N), 1).astype(jnp.float32)
    work = d2
    cols = []
    for _ in range(NS):
        m = jnp.min(work, axis=1, keepdims=True)
        cand = jnp.where(work == m, colid, jnp.float32(N))
        am = jnp.min(cand, axis=1, keepdims=True)
        cols.append(am)
        work = jnp.where(colid == am, jnp.inf, work)
    idxc = jnp.concatenate(cols, axis=1)
    idx_ref[0] = idxc

    sel = _onehot_rows(idxc)
    pr0 = _pr0_feat(sel, pc, pT, wp0T, bp0)
    sp_ref[0] = jnp.sum(pr0, axis=0, keepdims=True)
    ssp_ref[0] = jnp.sum(pr0 * pr0, axis=0, keepdims=True)


def _w0_feat(sel, q_ref, k_ref, p_r):
    g_k = jnp.dot(sel, k_ref[0], preferred_element_type=jnp.float32)
    return g_k - _rep_rows(q_ref[0]) + p_r


def _k2_body(q_ref, k_ref, idx_ref, p_ref, pT_ref, sp_ref, ssp_ref,
             gp_r, betap_r, wp0T, bp0, wp1T, bp1, s0_ref, ss0_ref):
    sel = _onehot_rows(idx_ref[0])
    p_r = _pos_feat(sel, p_ref, pT_ref, sp_ref, ssp_ref, gp_r, betap_r,
                    wp0T, bp0, wp1T, bp1)
    w0 = _w0_feat(sel, q_ref, k_ref, p_r)
    s0_ref[0] = jnp.sum(w0, axis=0, keepdims=True)
    ss0_ref[0] = jnp.sum(w0 * w0, axis=0, keepdims=True)


def _k3_body(q_ref, k_ref, idx_ref, p_ref, pT_ref, sp_ref, ssp_ref,
             gp_r, betap_r, wp0T, bp0, wp1T, bp1,
             s0_ref, ss0_ref, gw0_r, bw0_r, ww0T_r, bw0l_c,
             hT_ref, s1_ref, ss1_ref):
    sel = _onehot_rows(idx_ref[0])
    p_r = _pos_feat(sel, p_ref, pT_ref, sp_ref, ssp_ref, gp_r, betap_r,
                    wp0T, bp0, wp1T, bp1)
    w0 = _w0_feat(sel, q_ref, k_ref, p_r)
    scale0, shift0 = _bn_scale_shift(s0_ref, ss0_ref, gw0_r[...], bw0_r[...])
    w0n = jnp.maximum(w0 * scale0 + shift0, 0.0)
    hT = _dg(ww0T_r[...], w0n, 0, 1) + bw0l_c[...]
    hT_ref[0] = hT
    s1_ref[0] = jnp.sum(hT, axis=1, keepdims=True)
    ss1_ref[0] = jnp.sum(hT * hT, axis=1, keepdims=True)


def _k4_body(hT_ref, v_ref, idx_ref, p_ref, pT_ref, sp_ref, ssp_ref,
             gp_r, betap_r, wp0T, bp0, wp1T, bp1,
             s1_ref, ss1_ref, gw1_c, bw1_c, ww1T_r, bw1l_r,
             out_ref):
    scale1, shift1 = _bn_scale_shift(s1_ref, ss1_ref, gw1_c[...], bw1_c[...])
    hnT = jnp.maximum(hT_ref[0] * scale1 + shift1, 0.0)
    w1 = _dg(hnT, ww1T_r[...], 0, 0) + bw1l_r[...]
    w3 = w1.reshape(N, NS, CH)
    mx = jnp.max(w3, axis=1, keepdims=True)
    e = jnp.exp(w3 - mx)
    rinv = 1.0 / jnp.sum(e, axis=1, keepdims=True)
    wt = jnp.concatenate([e] * S, axis=2)

    sel = _onehot_rows(idx_ref[0])
    p_r = _pos_feat(sel, p_ref, pT_ref, sp_ref, ssp_ref, gp_r, betap_r,
                    wp0T, bp0, wp1T, bp1)
    g_v = jnp.dot(sel, v_ref[0], preferred_element_type=jnp.float32)
    a = (g_v + p_r).reshape(N, NS, C)
    acc = jnp.sum(a * wt, axis=1)
    rt = jnp.concatenate([rinv[:, 0, :]] * S, axis=1)
    out_ref[0] = acc * rt


def kernel(p, x, Wq, bq, Wk, bk, Wv, bv, Wp0, bp0, gp, betap, Wp1, bp1,
           gw0, bw0, Ww0, bw0l, gw1, bw1, Ww1, bw1l):
    f32 = jnp.float32
    xt = jnp.transpose(x, (0, 2, 1))
    pT = jnp.transpose(p, (0, 2, 1))
    wqT, wkT, wvT = Wq.T, Wk.T, Wv.T
    wp0T, wp1T, ww0T, ww1T = Wp0.T, Wp1.T, Ww0.T, Ww1.T
    r2 = lambda a: a.reshape(1, -1)
    rc = lambda a: a.reshape(-1, 1)
    bq2, bk2, bv2, bp02, bp12, bw1l2 = map(r2, (bq, bk, bv, bp0, bp1, bw1l))
    gp2, betap2, gw02, bw02 = map(r2, (gp, betap, gw0, bw0))
    bw0lc, gw1c, bw1c = map(rc, (bw0l, gw1, bw1))

    grid = (B,)
    full = lambda shape: pl.BlockSpec(shape, lambda b: (0,) * len(shape))
    bb = lambda *shape: pl.BlockSpec((1,) + shape, lambda b: (b,) + (0,) * len(shape))
    sd = jax.ShapeDtypeStruct

    q, k, v, idx, sp, ssp = pl.pallas_call(
        _k1_body,
        grid=grid,
        in_specs=[bb(N, C), bb(N, 3), bb(3, N),
                  full((C, C)), full((1, C)), full((C, C)), full((1, C)),
                  full((C, C)), full((1, C)), full((3, 3)), full((1, 3))],
        out_specs=[bb(N, C), bb(N, C), bb(N, C), bb(N, NS),
                   bb(1, 3), bb(1, 3)],
        out_shape=[sd((B, N, C), f32), sd((B, N, C), f32), sd((B, N, C), f32),
                   sd((B, N, NS), f32),
                   sd((B, 1, 3), f32), sd((B, 1, 3), f32)],
    )(xt, p, pT, wqT, bq2, wkT, bk2, wvT, bv2, wp0T, bp02)

    s0, ss0 = pl.pallas_call(
        _k2_body,
        grid=grid,
        in_specs=[bb(N, C), bb(N, C), bb(N, NS), bb(N, 3), bb(3, N),
                  full((B, 1, 3)), full((B, 1, 3)),
                  full((1, 3)), full((1, 3)), full((3, 3)), full((1, 3)),
                  full((3, C)), full((1, C))],
        out_specs=[bb(1, C), bb(1, C)],
        out_shape=[sd((B, 1, C), f32), sd((B, 1, C), f32)],
    )(q, k, idx, p, pT, sp, ssp, gp2, betap2, wp0T, bp02, wp1T, bp12)

    hT, s1, ss1 = pl.pallas_call(
        _k3_body,
        grid=grid,
        in_specs=[bb(N, C), bb(N, C), bb(N, NS), bb(N, 3), bb(3, N),
                  full((B, 1, 3)), full((B, 1, 3)),
                  full((1, 3)), full((1, 3)), full((3, 3)), full((1, 3)),
                  full((3, C)), full((1, C)),
                  full((B, 1, C)), full((B, 1, C)),
                  full((1, C)), full((1, C)), full((C, CH)), full((CH, 1))],
        out_specs=[bb(CH, RC), bb(CH, 1), bb(CH, 1)],
        out_shape=[sd((B, CH, RC), f32),
                   sd((B, CH, 1), f32), sd((B, CH, 1), f32)],
    )(q, k, idx, p, pT, sp, ssp, gp2, betap2, wp0T, bp02, wp1T, bp12,
      s0, ss0, gw02, bw02, ww0T, bw0lc)

    out = pl.pallas_call(
        _k4_body,
        grid=grid,
        in_specs=[bb(CH, RC), bb(N, C), bb(N, NS), bb(N, 3), bb(3, N),
                  full((B, 1, 3)), full((B, 1, 3)),
                  full((1, 3)), full((1, 3)), full((3, 3)), full((1, 3)),
                  full((3, C)), full((1, C)),
                  full((B, CH, 1)), full((B, CH, 1)),
                  full((CH, 1)), full((CH, 1)), full((CH, CH)), full((1, CH))],
        out_specs=pl.BlockSpec((1, N, C), lambda b: (b, 0, 0)),
        out_shape=sd((B, N, C), f32),
    )(hT, v, idx, p, pT, sp, ssp, gp2, betap2, wp0T, bp02, wp1T, bp12,
      s1, ss1, gw1c, bw1c, ww1T, bw1l2)

    return out


# speedup vs baseline: 1.5684x; 1.2974x over previous
import jax
import jax.numpy as jnp
from jax import lax
from jax.experimental import pallas as pl

B, N, NS = 8, 512, 16
C = 256
S = 8
CH = C // S
NCHUNK = 1
PC = N // NCHUNK
RC = PC * NS
CNT = float(B * N * NS)
EPS = 1e-5


def _onehot_rows(idxc):
    tgt = lax.broadcasted_iota(jnp.int32, (PC, NS, N), 2).astype(jnp.float32)
    sel = jnp.where(idxc[:, :, None] == tgt, 1.0, 0.0)
    return sel.reshape(RC, N)


def _rep_rows(a):
    d = a.shape[-1]
    return jnp.broadcast_to(a[:, None, :], (PC, NS, d)).reshape(RC, d)


def _bn_scale_shift(s_ref, ss_ref, gamma, beta):
    d = s_ref.shape[-1]
    ssum = jnp.sum(s_ref[...].reshape(B * NCHUNK, d), axis=0, keepdims=True)
    sssum = jnp.sum(ss_ref[...].reshape(B * NCHUNK, d), axis=0, keepdims=True)
    mean = ssum / CNT
    var = sssum / CNT - mean * mean
    scale = gamma / jnp.sqrt(var + EPS)
    shift = beta - mean * scale
    return scale, shift


def _pos_feat(pr0_ref, sp_ref, ssp_ref, gp_r, betap_r, wp1T_r, bp1_r):
    scalep, shiftp = _bn_scale_shift(sp_ref, ssp_ref, gp_r[...], betap_r[...])
    prn = jnp.maximum(pr0_ref[0] * scalep.reshape(3, 1) + shiftp.reshape(3, 1), 0.0)
    return lax.dot_general(prn, wp1T_r[...], (((0,), (0,)), ((), ())),
                           preferred_element_type=jnp.float32) + bp1_r[...]


def _k1_body(xt_ref, p_ref, pT_ref, wqT, bq, wkT, bk, wvT, bv, wp0T, bp0,
             q_ref, k_ref, v_ref, idx_ref, pr0_ref, sp_ref, ssp_ref):
    xtc = xt_ref[0]
    q_ref[0] = jnp.dot(xtc, wqT[...], preferred_element_type=jnp.float32) + bq[...]
    k_ref[0] = jnp.dot(xtc, wkT[...], preferred_element_type=jnp.float32) + bk[...]
    v_ref[0] = jnp.dot(xtc, wvT[...], preferred_element_type=jnp.float32) + bv[...]

    pc = p_ref[0]
    pT = pT_ref[0]
    dx = pc[:, 0:1] - pT[0:1, :]
    dy = pc[:, 1:2] - pT[1:2, :]
    dz = pc[:, 2:3] - pT[2:3, :]
    d2 = (dx * dx + dy * dy) + dz * dz

    colid = lax.broadcasted_iota(jnp.int32, (PC, N), 1).astype(jnp.float32)
    work = d2
    cols = []
    for _ in range(NS):
        m = jnp.min(work, axis=1, keepdims=True)
        cand = jnp.where(work == m, colid, jnp.float32(N))
        am = jnp.min(cand, axis=1, keepdims=True)
        cols.append(am)
        work = jnp.where(colid == am, jnp.inf, work)
    idxc = jnp.concatenate(cols, axis=1)
    idx_ref[0] = idxc

    sel = _onehot_rows(idxc)
    gp3 = lax.dot_general(sel, pT, (((1,), (1,)), ((), ())),
                          preferred_element_type=jnp.float32)
    prel = gp3 - _rep_rows(pc)
    pr0T = lax.dot_general(wp0T[...], prel, (((0,), (1,)), ((), ())),
                           preferred_element_type=jnp.float32) + bp0[...]
    pr0_ref[0] = pr0T
    sp_ref[0, 0] = jnp.sum(pr0T, axis=1, keepdims=True).reshape(1, 3)
    ssp_ref[0, 0] = jnp.sum(pr0T * pr0T, axis=1, keepdims=True).reshape(1, 3)


def _w0(q_ref, k_ref, idx_ref, p_r):
    sel = _onehot_rows(idx_ref[0])
    g_k = jnp.dot(sel, k_ref[0], preferred_element_type=jnp.float32)
    return g_k - _rep_rows(q_ref[0]) + p_r


def _k2_body(q_ref, k_ref, pr0_ref, idx_ref, sp_ref, ssp_ref,
             gp_r, betap_r, wp1T_r, bp1_r, s0_ref, ss0_ref):
    p_r = _pos_feat(pr0_ref, sp_ref, ssp_ref, gp_r, betap_r, wp1T_r, bp1_r)
    w0 = _w0(q_ref, k_ref, idx_ref, p_r)
    s0_ref[0, 0] = jnp.sum(w0, axis=0, keepdims=True)
    ss0_ref[0, 0] = jnp.sum(w0 * w0, axis=0, keepdims=True)


def _k3_body(q_ref, k_ref, pr0_ref, idx_ref, sp_ref, ssp_ref,
             gp_r, betap_r, wp1T_r, bp1_r,
             s0_ref, ss0_ref, gw0_r, bw0_r, ww0T_r, bw0l_r,
             h_ref, s1_ref, ss1_ref):
    p_r = _pos_feat(pr0_ref, sp_ref, ssp_ref, gp_r, betap_r, wp1T_r, bp1_r)
    w0 = _w0(q_ref, k_ref, idx_ref, p_r)
    scale0, shift0 = _bn_scale_shift(s0_ref, ss0_ref, gw0_r[...], bw0_r[...])
    w0n = jnp.maximum(w0 * scale0 + shift0, 0.0)
    h = jnp.dot(w0n, ww0T_r[...], preferred_element_type=jnp.float32) + bw0l_r[...]
    h_ref[0] = h
    s1_ref[0, 0] = jnp.sum(h, axis=0, keepdims=True)
    ss1_ref[0, 0] = jnp.sum(h * h, axis=0, keepdims=True)


def _k4_body(h_ref, v_ref, pr0_ref, idx_ref, sp_ref, ssp_ref,
             gp_r, betap_r, wp1T_r, bp1_r,
             s1_ref, ss1_ref, gw1_r, bw1_r, ww1T_r, bw1l_r,
             out_ref):
    scale1, shift1 = _bn_scale_shift(s1_ref, ss1_ref, gw1_r[...], bw1_r[...])
    hn = jnp.maximum(h_ref[0] * scale1 + shift1, 0.0)
    w1 = jnp.dot(hn, ww1T_r[...], preferred_element_type=jnp.float32) + bw1l_r[...]
    w3 = w1.reshape(PC, NS, CH)
    mx = jnp.max(w3, axis=1, keepdims=True)
    e = jnp.exp(w3 - mx)
    rinv = 1.0 / jnp.sum(e, axis=1, keepdims=True)
    wt = jnp.concatenate([e] * S, axis=2)

    p_r = _pos_feat(pr0_ref, sp_ref, ssp_ref, gp_r, betap_r, wp1T_r, bp1_r)
    sel = _onehot_rows(idx_ref[0])
    g_v = jnp.dot(sel, v_ref[0], preferred_element_type=jnp.float32)
    a = (g_v + p_r).reshape(PC, NS, C)
    acc = jnp.sum(a * wt, axis=1)
    rt = jnp.concatenate([rinv[:, 0, :]] * S, axis=1)
    out_ref[0] = acc * rt


def kernel(p, x, Wq, bq, Wk, bk, Wv, bv, Wp0, bp0, gp, betap, Wp1, bp1,
           gw0, bw0, Ww0, bw0l, gw1, bw1, Ww1, bw1l):
    f32 = jnp.float32
    xt = jnp.transpose(x, (0, 2, 1))
    pT = jnp.transpose(p, (0, 2, 1))
    wqT, wkT, wvT = Wq.T, Wk.T, Wv.T
    wp0T, wp1T, ww0T, ww1T = Wp0.T, Wp1.T, Ww0.T, Ww1.T
    r2 = lambda a: a.reshape(1, -1)
    bq2, bk2, bv2, bp12, bw0l2, bw1l2 = map(r2, (bq, bk, bv, bp1, bw0l, bw1l))
    bp02 = bp0.reshape(-1, 1)
    gp2, betap2, gw02, bw02, gw12, bw12 = map(r2, (gp, betap, gw0, bw0, gw1, bw1))

    grid = (B, NCHUNK)
    full = lambda shape: pl.BlockSpec(shape, lambda b, c: (0,) * len(shape))
    bc = lambda *shape: pl.BlockSpec((1,) + shape, lambda b, c: (b, c) + (0,) * (len(shape) - 1))
    bonly = lambda *shape: pl.BlockSpec((1,) + shape, lambda b, c: (b,) + (0,) * len(shape))
    stat = lambda d: pl.BlockSpec((1, 1, 1, d), lambda b, c: (b, c, 0, 0))
    sd = jax.ShapeDtypeStruct

    q, k, v, idx, pr0, sp, ssp = pl.pallas_call(
        _k1_body,
        grid=grid,
        in_specs=[bc(PC, C), bc(PC, 3), bonly(3, N),
                  full((C, C)), full((1, C)), full((C, C)), full((1, C)),
                  full((C, C)), full((1, C)), full((3, 3)), full((3, 1))],
        out_specs=[bc(PC, C), bc(PC, C), bc(PC, C), bc(PC, NS), bonly(3, N * NS),
                   stat(3), stat(3)],
        out_shape=[sd((B, N, C), f32), sd((B, N, C), f32), sd((B, N, C), f32),
                   sd((B, N, NS), f32), sd((B, 3, N * NS), f32),
                   sd((B, NCHUNK, 1, 3), f32), sd((B, NCHUNK, 1, 3), f32)],
    )(xt, p, pT, wqT, bq2, wkT, bk2, wvT, bv2, wp0T, bp02)

    s0, ss0 = pl.pallas_call(
        _k2_body,
        grid=grid,
        in_specs=[bc(PC, C), bonly(N, C), bonly(3, N * NS), bc(PC, NS),
                  full((B, NCHUNK, 1, 3)), full((B, NCHUNK, 1, 3)),
                  full((1, 3)), full((1, 3)), full((3, C)), full((1, C))],
        out_specs=[stat(C), stat(C)],
        out_shape=[sd((B, NCHUNK, 1, C), f32), sd((B, NCHUNK, 1, C), f32)],
    )(q, k, pr0, idx, sp, ssp, gp2, betap2, wp1T, bp12)

    h, s1, ss1 = pl.pallas_call(
        _k3_body,
        grid=grid,
        in_specs=[bc(PC, C), bonly(N, C), bonly(3, N * NS), bc(PC, NS),
                  full((B, NCHUNK, 1, 3)), full((B, NCHUNK, 1, 3)),
                  full((1, 3)), full((1, 3)), full((3, C)), full((1, C)),
                  full((B, NCHUNK, 1, C)), full((B, NCHUNK, 1, C)),
                  full((1, C)), full((1, C)), full((C, CH)), full((1, CH))],
        out_specs=[bc(RC, CH), stat(CH), stat(CH)],
        out_shape=[sd((B, N * NS, CH), f32),
                   sd((B, NCHUNK, 1, CH), f32), sd((B, NCHUNK, 1, CH), f32)],
    )(q, k, pr0, idx, sp, ssp, gp2, betap2, wp1T, bp12,
      s0, ss0, gw02, bw02, ww0T, bw0l2)

    out = pl.pallas_call(
        _k4_body,
        grid=grid,
        in_specs=[bc(RC, CH), bonly(N, C), bonly(3, N * NS), bc(PC, NS),
                  full((B, NCHUNK, 1, 3)), full((B, NCHUNK, 1, 3)),
                  full((1, 3)), full((1, 3)), full((3, C)), full((1, C)),
                  full((B, NCHUNK, 1, CH)), full((B, NCHUNK, 1, CH)),
                  full((1, CH)), full((1, CH)), full((CH, CH)), full((1, CH))],
        out_specs=[bc(PC, C)],
        out_shape=[sd((B, N, C), f32)],
    )(h, v, pr0, idx, sp, ssp, gp2, betap2, wp1T, bp12,
      s1, ss1, gw12, bw12, ww1T, bw1l2)[0]

    return out


# speedup vs baseline: 1.5804x; 1.0077x over previous
import jax
import jax.numpy as jnp
from jax import lax
from jax.experimental import pallas as pl

B, N, NS = 8, 512, 16
C = 256
S = 8
CH = C // S
NCHUNK = 1
PC = N // NCHUNK
RC = PC * NS
CNT = float(B * N * NS)
EPS = 1e-5


def _onehot_rows(idxc):
    tgt = lax.broadcasted_iota(jnp.int32, (PC, NS, N), 2).astype(jnp.float32)
    sel = jnp.where(idxc[:, :, None] == tgt, 1.0, 0.0)
    return sel.reshape(RC, N)


def _rep_rows(a):
    d = a.shape[-1]
    return jnp.broadcast_to(a[:, None, :], (PC, NS, d)).reshape(RC, d)


def _bn_scale_shift(s_ref, ss_ref, gamma, beta):
    d = s_ref.shape[-1]
    ssum = jnp.sum(s_ref[...].reshape(B * NCHUNK, d), axis=0, keepdims=True)
    sssum = jnp.sum(ss_ref[...].reshape(B * NCHUNK, d), axis=0, keepdims=True)
    mean = ssum / CNT
    var = sssum / CNT - mean * mean
    scale = gamma / jnp.sqrt(var + EPS)
    shift = beta - mean * scale
    return scale, shift


def _pos_feat(pr0_ref, sp_ref, ssp_ref, gp_r, betap_r, wp1T_r, bp1_r):
    scalep, shiftp = _bn_scale_shift(sp_ref, ssp_ref, gp_r[...], betap_r[...])
    prn = jnp.maximum(pr0_ref[0] * scalep.reshape(3, 1) + shiftp.reshape(3, 1), 0.0)
    return lax.dot_general(prn, wp1T_r[...], (((0,), (0,)), ((), ())),
                           preferred_element_type=jnp.float32) + bp1_r[...]


def _k1_body(xt_ref, p_ref, pT_ref, wqT, bq, wkT, bk, wvT, bv, wp0T, bp0,
             q_ref, k_ref, v_ref, idx_ref, pr0_ref, sp_ref, ssp_ref):
    xtc = xt_ref[0]
    q_ref[0] = jnp.dot(xtc, wqT[...], preferred_element_type=jnp.float32) + bq[...]
    k_ref[0] = jnp.dot(xtc, wkT[...], preferred_element_type=jnp.float32) + bk[...]
    v_ref[0] = jnp.dot(xtc, wvT[...], preferred_element_type=jnp.float32) + bv[...]

    pc = p_ref[0]
    pT = pT_ref[0]
    dx = pc[:, 0:1] - pT[0:1, :]
    dy = pc[:, 1:2] - pT[1:2, :]
    dz = pc[:, 2:3] - pT[2:3, :]
    d2 = (dx * dx + dy * dy) + dz * dz

    colid = lax.broadcasted_iota(jnp.int32, (PC, N), 1).astype(jnp.float32)
    work = d2
    cols = []
    for _ in range(NS):
        m = jnp.min(work, axis=1, keepdims=True)
        cand = jnp.where(work == m, colid, jnp.float32(N))
        am = jnp.min(cand, axis=1, keepdims=True)
        cols.append(am)
        work = jnp.where(colid == am, jnp.inf, work)
    idxc = jnp.concatenate(cols, axis=1)
    idx_ref[0] = idxc

    sel = _onehot_rows(idxc)
    gp3 = lax.dot_general(sel, pT, (((1,), (1,)), ((), ())),
                          preferred_element_type=jnp.float32)
    prel = gp3 - _rep_rows(pc)
    pr0T = lax.dot_general(wp0T[...], prel, (((0,), (1,)), ((), ())),
                           preferred_element_type=jnp.float32) + bp0[...]
    pr0_ref[0] = pr0T
    sp_ref[0, 0] = jnp.sum(pr0T, axis=1, keepdims=True).reshape(1, 3)
    ssp_ref[0, 0] = jnp.sum(pr0T * pr0T, axis=1, keepdims=True).reshape(1, 3)


def _w0(q_ref, k_ref, idx_ref, p_r):
    sel = _onehot_rows(idx_ref[0])
    g_k = jnp.dot(sel, k_ref[0], preferred_element_type=jnp.float32)
    return g_k - _rep_rows(q_ref[0]) + p_r


def _k2_body(q_ref, k_ref, pr0_ref, idx_ref, sp_ref, ssp_ref,
             gp_r, betap_r, wp1T_r, bp1_r, s0_ref, ss0_ref):
    p_r = _pos_feat(pr0_ref, sp_ref, ssp_ref, gp_r, betap_r, wp1T_r, bp1_r)
    w0 = _w0(q_ref, k_ref, idx_ref, p_r)
    s0_ref[0, 0] = jnp.sum(w0, axis=0, keepdims=True)
    ss0_ref[0, 0] = jnp.sum(w0 * w0, axis=0, keepdims=True)


def _k3_body(q_ref, k_ref, pr0_ref, idx_ref, sp_ref, ssp_ref,
             gp_r, betap_r, wp1T_r, bp1_r,
             s0_ref, ss0_ref, gw0_r, bw0_r, ww0T_r, bw0l_r,
             h_ref, s1_ref, ss1_ref):
    p_r = _pos_feat(pr0_ref, sp_ref, ssp_ref, gp_r, betap_r, wp1T_r, bp1_r)
    w0 = _w0(q_ref, k_ref, idx_ref, p_r)
    scale0, shift0 = _bn_scale_shift(s0_ref, ss0_ref, gw0_r[...], bw0_r[...])
    w0n = jnp.maximum(w0 * scale0 + shift0, 0.0)
    hT = lax.dot_general(ww0T_r[...], w0n, (((0,), (1,)), ((), ())),
                         preferred_element_type=jnp.float32) + bw0l_r[...]
    h_ref[0] = hT
    s1_ref[0, 0] = jnp.sum(hT, axis=1, keepdims=True).reshape(1, CH)
    ss1_ref[0, 0] = jnp.sum(hT * hT, axis=1, keepdims=True).reshape(1, CH)


def _k4_body(h_ref, v_ref, pr0_ref, idx_ref, sp_ref, ssp_ref,
             gp_r, betap_r, wp1T_r, bp1_r,
             s1_ref, ss1_ref, gw1_r, bw1_r, ww1T_r, bw1l_r,
             out_ref):
    scale1, shift1 = _bn_scale_shift(s1_ref, ss1_ref, gw1_r[...], bw1_r[...])
    hnT = jnp.maximum(h_ref[0] * scale1.reshape(CH, 1) + shift1.reshape(CH, 1), 0.0)
    w1 = lax.dot_general(hnT, ww1T_r[...], (((0,), (0,)), ((), ())),
                         preferred_element_type=jnp.float32) + bw1l_r[...]
    w3 = w1.reshape(PC, NS, CH)
    mx = jnp.max(w3, axis=1, keepdims=True)
    e = jnp.exp(w3 - mx)
    rinv = 1.0 / jnp.sum(e, axis=1, keepdims=True)
    wt = jnp.concatenate([e] * S, axis=2)

    p_r = _pos_feat(pr0_ref, sp_ref, ssp_ref, gp_r, betap_r, wp1T_r, bp1_r)
    sel = _onehot_rows(idx_ref[0])
    g_v = jnp.dot(sel, v_ref[0], preferred_element_type=jnp.float32)
    a = (g_v + p_r).reshape(PC, NS, C)
    acc = jnp.sum(a * wt, axis=1)
    rt = jnp.concatenate([rinv[:, 0, :]] * S, axis=1)
    out_ref[0] = acc * rt


def kernel(p, x, Wq, bq, Wk, bk, Wv, bv, Wp0, bp0, gp, betap, Wp1, bp1,
           gw0, bw0, Ww0, bw0l, gw1, bw1, Ww1, bw1l):
    f32 = jnp.float32
    xt = jnp.transpose(x, (0, 2, 1))
    pT = jnp.transpose(p, (0, 2, 1))
    wqT, wkT, wvT = Wq.T, Wk.T, Wv.T
    wp0T, wp1T, ww0T, ww1T = Wp0.T, Wp1.T, Ww0.T, Ww1.T
    r2 = lambda a: a.reshape(1, -1)
    bq2, bk2, bv2, bp12, bw1l2 = map(r2, (bq, bk, bv, bp1, bw1l))
    bw0l2 = bw0l.reshape(-1, 1)
    bp02 = bp0.reshape(-1, 1)
    gp2, betap2, gw02, bw02, gw12, bw12 = map(r2, (gp, betap, gw0, bw0, gw1, bw1))

    grid = (B, NCHUNK)
    full = lambda shape: pl.BlockSpec(shape, lambda b, c: (0,) * len(shape))
    bc = lambda *shape: pl.BlockSpec((1,) + shape, lambda b, c: (b, c) + (0,) * (len(shape) - 1))
    bonly = lambda *shape: pl.BlockSpec((1,) + shape, lambda b, c: (b,) + (0,) * len(shape))
    stat = lambda d: pl.BlockSpec((1, 1, 1, d), lambda b, c: (b, c, 0, 0))
    sd = jax.ShapeDtypeStruct

    q, k, v, idx, pr0, sp, ssp = pl.pallas_call(
        _k1_body,
        grid=grid,
        in_specs=[bc(PC, C), bc(PC, 3), bonly(3, N),
                  full((C, C)), full((1, C)), full((C, C)), full((1, C)),
                  full((C, C)), full((1, C)), full((3, 3)), full((3, 1))],
        out_specs=[bc(PC, C), bc(PC, C), bc(PC, C), bc(PC, NS), bonly(3, N * NS),
                   stat(3), stat(3)],
        out_shape=[sd((B, N, C), f32), sd((B, N, C), f32), sd((B, N, C), f32),
                   sd((B, N, NS), f32), sd((B, 3, N * NS), f32),
                   sd((B, NCHUNK, 1, 3), f32), sd((B, NCHUNK, 1, 3), f32)],
    )(xt, p, pT, wqT, bq2, wkT, bk2, wvT, bv2, wp0T, bp02)

    s0, ss0 = pl.pallas_call(
        _k2_body,
        grid=grid,
        in_specs=[bc(PC, C), bonly(N, C), bonly(3, N * NS), bc(PC, NS),
                  full((B, NCHUNK, 1, 3)), full((B, NCHUNK, 1, 3)),
                  full((1, 3)), full((1, 3)), full((3, C)), full((1, C))],
        out_specs=[stat(C), stat(C)],
        out_shape=[sd((B, NCHUNK, 1, C), f32), sd((B, NCHUNK, 1, C), f32)],
    )(q, k, pr0, idx, sp, ssp, gp2, betap2, wp1T, bp12)

    h, s1, ss1 = pl.pallas_call(
        _k3_body,
        grid=grid,
        in_specs=[bc(PC, C), bonly(N, C), bonly(3, N * NS), bc(PC, NS),
                  full((B, NCHUNK, 1, 3)), full((B, NCHUNK, 1, 3)),
                  full((1, 3)), full((1, 3)), full((3, C)), full((1, C)),
                  full((B, NCHUNK, 1, C)), full((B, NCHUNK, 1, C)),
                  full((1, C)), full((1, C)), full((C, CH)), full((CH, 1))],
        out_specs=[bonly(CH, N * NS), stat(CH), stat(CH)],
        out_shape=[sd((B, CH, N * NS), f32),
                   sd((B, NCHUNK, 1, CH), f32), sd((B, NCHUNK, 1, CH), f32)],
    )(q, k, pr0, idx, sp, ssp, gp2, betap2, wp1T, bp12,
      s0, ss0, gw02, bw02, ww0T, bw0l2)

    out = pl.pallas_call(
        _k4_body,
        grid=grid,
        in_specs=[bonly(CH, N * NS), bonly(N, C), bonly(3, N * NS), bc(PC, NS),
                  full((B, NCHUNK, 1, 3)), full((B, NCHUNK, 1, 3)),
                  full((1, 3)), full((1, 3)), full((3, C)), full((1, C)),
                  full((B, NCHUNK, 1, CH)), full((B, NCHUNK, 1, CH)),
                  full((1, CH)), full((1, CH)), full((CH, CH)), full((1, CH))],
        out_specs=[bc(PC, C)],
        out_shape=[sd((B, N, C), f32)],
    )(h, v, pr0, idx, sp, ssp, gp2, betap2, wp1T, bp12,
      s1, ss1, gw12, bw12, ww1T, bw1l2)[0]

    return out
